# Initial kernel scaffold; baseline (speedup 1.0000x reference)
import jax
import jax.numpy as jnp
from jax.experimental import pallas as pl


def kernel(x, A, B, adapter_ids):
    raise NotImplementedError("write your pallas kernel here")


# speedup vs baseline: 1.7695x; 1.7695x over previous
import jax
import jax.numpy as jnp
from jax.experimental import pallas as pl
from jax.experimental.pallas import tpu as pltpu

_RANK = 64
_SCALE = 1.0 / _RANK


def _body(ids_ref, x_ref, b_ref, a_ref, o_ref):
    xb = x_ref[0].astype(jnp.bfloat16)
    t = jnp.dot(xb, b_ref[0].astype(jnp.bfloat16),
                preferred_element_type=jnp.float32)
    o_ref[0] = jnp.dot(t.astype(jnp.bfloat16), a_ref[...].astype(jnp.bfloat16),
                       preferred_element_type=jnp.float32) * _SCALE


def kernel(x, A, B, adapter_ids):
    Bt, S, H = x.shape
    R, OUT = A.shape
    n_adapters = R // _RANK
    B3 = jnp.transpose(B.reshape(H, n_adapters, _RANK), (1, 0, 2))
    TILE = 512
    grid = (Bt, S // TILE)
    grid_spec = pltpu.PrefetchScalarGridSpec(
        num_scalar_prefetch=1,
        grid=grid,
        in_specs=[
            pl.BlockSpec((1, TILE, H), lambda b, i, ids: (b, i, 0)),
            pl.BlockSpec((1, H, _RANK), lambda b, i, ids: (ids[b], 0, 0)),
            pl.BlockSpec((_RANK, OUT), lambda b, i, ids: (ids[b], 0)),
        ],
        out_specs=pl.BlockSpec((1, TILE, OUT), lambda b, i, ids: (b, i, 0)),
    )
    return pl.pallas_call(
        _body,
        grid_spec=grid_spec,
        out_shape=jax.ShapeDtypeStruct((Bt, S, OUT), jnp.float32),
    )(adapter_ids, x, B3, A)


# speedup vs baseline: 1.7707x; 1.0007x over previous
import jax
import jax.numpy as jnp
from jax.experimental import pallas as pl
from jax.experimental.pallas import tpu as pltpu

_RANK = 64
_SCALE = 1.0 / _RANK


def _body(ids_ref, x_ref, b_ref, a_ref, o_ref):
    xb = x_ref[0].astype(jnp.bfloat16)
    t = jnp.dot(xb, b_ref[0].astype(jnp.bfloat16),
                preferred_element_type=jnp.float32)
    o_ref[0] = jnp.dot(t.astype(jnp.bfloat16), a_ref[...].astype(jnp.bfloat16),
                       preferred_element_type=jnp.float32) * _SCALE


def kernel(x, A, B, adapter_ids):
    Bt, S, H = x.shape
    R, OUT = A.shape
    n_adapters = R // _RANK
    B3 = jnp.transpose(B.reshape(H, n_adapters, _RANK), (1, 0, 2))
    TILE = 512
    grid = (Bt, S // TILE)
    grid_spec = pltpu.PrefetchScalarGridSpec(
        num_scalar_prefetch=1,
        grid=grid,
        in_specs=[
            pl.BlockSpec((1, TILE, H), lambda b, i, ids: (b, i, 0)),
            pl.BlockSpec((1, H, _RANK), lambda b, i, ids: (ids[b], 0, 0)),
            pl.BlockSpec((_RANK, OUT), lambda b, i, ids: (ids[b], 0)),
        ],
        out_specs=pl.BlockSpec((1, TILE, OUT), lambda b, i, ids: (b, i, 0)),
    )
    return pl.pallas_call(
        _body,
        grid_spec=grid_spec,
        out_shape=jax.ShapeDtypeStruct((Bt, S, OUT), jnp.float32),
        compiler_params=pltpu.CompilerParams(
            dimension_semantics=("parallel", "parallel")),
    )(adapter_ids, x, B3, A)


# speedup vs baseline: 1.7899x; 1.0108x over previous
import jax
import jax.numpy as jnp
from jax.experimental import pallas as pl
from jax.experimental.pallas import tpu as pltpu

_RANK = 64
_SCALE = 1.0 / _RANK


def _body(ids_ref, x_ref, b_ref, a_ref, o_ref):
    xb = x_ref[0].astype(jnp.bfloat16)
    t = jnp.dot(xb, b_ref[0], preferred_element_type=jnp.float32)
    o_ref[0] = jnp.dot(t.astype(jnp.bfloat16), a_ref[...],
                       preferred_element_type=jnp.float32)


def kernel(x, A, B, adapter_ids):
    Bt, S, H = x.shape
    R, OUT = A.shape
    n_adapters = R // _RANK
    B3 = jnp.transpose(B.reshape(H, n_adapters, _RANK), (1, 0, 2))
    B3 = B3.astype(jnp.bfloat16)
    A = (A * _SCALE).astype(jnp.bfloat16)
    TILE = 512
    grid = (Bt, S // TILE)
    grid_spec = pltpu.PrefetchScalarGridSpec(
        num_scalar_prefetch=1,
        grid=grid,
        in_specs=[
            pl.BlockSpec((1, TILE, H), lambda b, i, ids: (b, i, 0)),
            pl.BlockSpec((1, H, _RANK), lambda b, i, ids: (ids[b], 0, 0)),
            pl.BlockSpec((_RANK, OUT), lambda b, i, ids: (ids[b], 0)),
        ],
        out_specs=pl.BlockSpec((1, TILE, OUT), lambda b, i, ids: (b, i, 0)),
    )
    return pl.pallas_call(
        _body,
        grid_spec=grid_spec,
        out_shape=jax.ShapeDtypeStruct((Bt, S, OUT), jnp.float32),
        compiler_params=pltpu.CompilerParams(
            dimension_semantics=("parallel", "parallel")),
    )(adapter_ids, x, B3, A)


# speedup vs baseline: 1.8643x; 1.0416x over previous
import jax
import jax.numpy as jnp
from jax.experimental import pallas as pl
from jax.experimental.pallas import tpu as pltpu

_RANK = 64
_SCALE = 1.0 / _RANK


def _body(ids_ref, x_ref, b_ref, a_ref, o_ref):
    xb = x_ref[0].astype(jnp.bfloat16)
    t = jnp.dot(xb, b_ref[0], preferred_element_type=jnp.float32)
    o_ref[0] = jnp.dot(t.astype(jnp.bfloat16), a_ref[...],
                       preferred_element_type=jnp.float32)


def kernel(x, A, B, adapter_ids):
    Bt, S, H = x.shape
    R, OUT = A.shape
    n_adapters = R // _RANK
    B3 = jnp.transpose(B.reshape(H, n_adapters, _RANK), (1, 0, 2))
    B3 = B3.astype(jnp.bfloat16)
    A = (A * _SCALE).astype(jnp.bfloat16)
    TILE = 256
    grid = (Bt, S // TILE)
    grid_spec = pltpu.PrefetchScalarGridSpec(
        num_scalar_prefetch=1,
        grid=grid,
        in_specs=[
            pl.BlockSpec((1, TILE, H), lambda b, i, ids: (b, i, 0)),
            pl.BlockSpec((1, H, _RANK), lambda b, i, ids: (ids[b], 0, 0)),
            pl.BlockSpec((_RANK, OUT), lambda b, i, ids: (ids[b], 0)),
        ],
        out_specs=pl.BlockSpec((1, TILE, OUT), lambda b, i, ids: (b, i, 0)),
    )
    return pl.pallas_call(
        _body,
        grid_spec=grid_spec,
        out_shape=jax.ShapeDtypeStruct((Bt, S, OUT), jnp.float32),
        compiler_params=pltpu.CompilerParams(
            dimension_semantics=("parallel", "parallel")),
    )(adapter_ids, x, B3, A)
